# initial kernel scaffold (unmeasured)
import functools

import jax
import jax.numpy as jnp
from jax import lax
from jax.experimental import pallas as pl
from jax.experimental.pallas import tpu as pltpu

N_DEV = 4
M_BLK = 1024
K_BLK = 1024


def kernel(x, w_mat, scale_x, scale_w):
    k_glob, k_per = x.shape
    _, n = w_mat.shape
    m_per = k_glob // N_DEV

    def body(x_ref, w_ref, sx_ref, sw_ref, out_ref,
             xbuf_ref, send_sems, recv_sems):
        me = lax.axis_index("i")

        barrier_sem = pltpu.get_barrier_semaphore()
        for d in range(1, N_DEV):
            pl.semaphore_signal(
                barrier_sem, inc=1,
                device_id=((me + d) % N_DEV,),
                device_id_type=pl.DeviceIdType.MESH,
            )
        pl.semaphore_wait(barrier_sem, N_DEV - 1)

        rdmas = []
        for d in range(1, N_DEV):
            t = (me + d) % N_DEV
            rdma = pltpu.make_async_remote_copy(
                src_ref=x_ref.at[pl.ds(t * m_per, m_per), :],
                dst_ref=xbuf_ref.at[d - 1],
                send_sem=send_sems.at[d - 1],
                recv_sem=recv_sems.at[d - 1],
                device_id=(t,),
                device_id_type=pl.DeviceIdType.MESH,
            )
            rdma.start()
            rdmas.append(rdma)

        acc = jnp.dot(
            x_ref[pl.ds(me * m_per, m_per), :],
            w_ref[pl.ds(me * k_per, k_per), :],
            preferred_element_type=jnp.float32,
        )

        for d in range(1, N_DEV):
            rdmas[d - 1].wait_recv()
            s = (me - d) % N_DEV
            acc = acc + jnp.dot(
                xbuf_ref[d - 1],
                w_ref[pl.ds(s * k_per, k_per), :],
                preferred_element_type=jnp.float32,
            )

        for d in range(1, N_DEV):
            rdmas[d - 1].wait_send()

        scale = sx_ref[0] * sw_ref[0]
        out_ref[:, :] = jnp.maximum(acc * scale, 0.0)

        @functools.partial(
            pl.run_scoped, exit_sem=pltpu.SemaphoreType.REGULAR
        )
        def _(exit_sem):
            for d in range(1, N_DEV):
                pl.semaphore_signal(
                    exit_sem, inc=1,
                    device_id=((me + d) % N_DEV,),
                    device_id_type=pl.DeviceIdType.MESH,
                )
            pl.semaphore_wait(exit_sem, N_DEV - 1)

    return pl.pallas_call(
        body,
        out_shape=jax.ShapeDtypeStruct((m_per, n), jnp.float32),
        in_specs=[
            pl.BlockSpec(memory_space=pltpu.VMEM),
            pl.BlockSpec(memory_space=pltpu.VMEM),
            pl.BlockSpec(memory_space=pltpu.SMEM),
            pl.BlockSpec(memory_space=pltpu.SMEM),
        ],
        out_specs=pl.BlockSpec(memory_space=pltpu.VMEM),
        scratch_shapes=[
            pltpu.VMEM((N_DEV - 1, m_per, k_per), x.dtype),
            pltpu.SemaphoreType.DMA((N_DEV - 1,)),
            pltpu.SemaphoreType.DMA((N_DEV - 1,)),
        ],
        compiler_params=pltpu.CompilerParams(collective_id=0),
    )(x, w_mat, scale_x, scale_w)


# baseline (device time: 46569 ns/iter reference)
import functools

import jax
import jax.numpy as jnp
from jax import lax
from jax.experimental import pallas as pl
from jax.experimental.pallas import tpu as pltpu

N_DEV = 4
W_CHUNKS = 8


def kernel(x, w_mat, scale_x, scale_w):
    k_glob, k_per = x.shape
    _, n = w_mat.shape
    m_per = k_glob // N_DEV
    wck = k_glob // W_CHUNKS

    def body(x_ref, w_ref, sx_ref, sw_ref, out_ref,
             xstage_ref, wstage_ref, xsend_ref, w8_ref, xbuf_ref,
             load_sem, send_sems, recv_sems):
        me = lax.axis_index("i")

        barrier_sem = pltpu.get_barrier_semaphore()
        for d in range(1, N_DEV):
            pl.semaphore_signal(
                barrier_sem, inc=1,
                device_id=((me + d) % N_DEV,),
                device_id_type=pl.DeviceIdType.MESH,
            )
        pl.semaphore_wait(barrier_sem, N_DEV - 1)

        rdmas = []
        for d in range(1, N_DEV):
            t = (me + d) % N_DEV
            cp = pltpu.make_async_copy(
                x_ref.at[pl.ds(t * m_per, m_per), :], xstage_ref, load_sem
            )
            cp.start()
            cp.wait()
            xsend_ref[d - 1] = xstage_ref[...].astype(jnp.float8_e4m3fn)
            rdma = pltpu.make_async_remote_copy(
                src_ref=xsend_ref.at[d - 1],
                dst_ref=xbuf_ref.at[d - 1],
                send_sem=send_sems.at[d - 1],
                recv_sem=recv_sems.at[d - 1],
                device_id=(t,),
                device_id_type=pl.DeviceIdType.MESH,
            )
            rdma.start()
            rdmas.append(rdma)

        cp = pltpu.make_async_copy(
            x_ref.at[pl.ds(me * m_per, m_per), :], xstage_ref, load_sem
        )
        cp.start()
        cp.wait()
        xsend_ref[N_DEV - 1] = xstage_ref[...].astype(jnp.float8_e4m3fn)

        for j in range(W_CHUNKS):
            cp = pltpu.make_async_copy(
                w_ref.at[pl.ds(j * wck, wck), :], wstage_ref, load_sem
            )
            cp.start()
            cp.wait()
            w8_ref[pl.ds(j * wck, wck), :] = (
                wstage_ref[...].astype(jnp.float8_e5m2)
            )

        out_ref[:, :] = jnp.dot(
            xsend_ref[N_DEV - 1],
            w8_ref[pl.ds(me * k_per, k_per), :],
            preferred_element_type=jnp.float32,
        )

        for d in range(1, N_DEV):
            rdmas[d - 1].wait_recv()
            s = (me - d) % N_DEV
            out_ref[:, :] += jnp.dot(
                xbuf_ref[d - 1],
                w8_ref[pl.ds(s * k_per, k_per), :],
                preferred_element_type=jnp.float32,
            )

        for d in range(1, N_DEV):
            rdmas[d - 1].wait_send()

        scale = sx_ref[0] * sw_ref[0]
        out_ref[:, :] = jnp.maximum(out_ref[:, :] * scale, 0.0)

        @functools.partial(
            pl.run_scoped, exit_sem=pltpu.SemaphoreType.REGULAR
        )
        def _(exit_sem):
            for d in range(1, N_DEV):
                pl.semaphore_signal(
                    exit_sem, inc=1,
                    device_id=((me + d) % N_DEV,),
                    device_id_type=pl.DeviceIdType.MESH,
                )
            pl.semaphore_wait(exit_sem, N_DEV - 1)

    return pl.pallas_call(
        body,
        out_shape=jax.ShapeDtypeStruct((m_per, n), jnp.float32),
        in_specs=[
            pl.BlockSpec(memory_space=pl.ANY),
            pl.BlockSpec(memory_space=pl.ANY),
            pl.BlockSpec(memory_space=pltpu.SMEM),
            pl.BlockSpec(memory_space=pltpu.SMEM),
        ],
        out_specs=pl.BlockSpec(memory_space=pltpu.VMEM),
        scratch_shapes=[
            pltpu.VMEM((m_per, k_per), jnp.float32),
            pltpu.VMEM((wck, n), jnp.float32),
            pltpu.VMEM((N_DEV, m_per, k_per), jnp.float8_e4m3fn),
            pltpu.VMEM((k_glob, n), jnp.float8_e5m2),
            pltpu.VMEM((N_DEV - 1, m_per, k_per), jnp.float8_e4m3fn),
            pltpu.SemaphoreType.DMA,
            pltpu.SemaphoreType.DMA((N_DEV - 1,)),
            pltpu.SemaphoreType.DMA((N_DEV - 1,)),
        ],
        compiler_params=pltpu.CompilerParams(collective_id=0),
    )(x, w_mat, scale_x, scale_w)
